# Initial kernel scaffold; baseline (speedup 1.0000x reference)
import jax
import jax.numpy as jnp
from jax.experimental import pallas as pl


def kernel(x, W_qk):
    raise NotImplementedError("write your pallas kernel here")


# speedup vs baseline: 3.9363x; 3.9363x over previous
import functools

import jax
import jax.numpy as jnp
from jax.experimental import pallas as pl
from jax.experimental.pallas import tpu as pltpu

_H = 8
_NB = 5
_RB = 256


def _attn_rows(x_ref, wq_ref, wk_ref, kt, r, scale):
    xr = x_ref[0, r * _RB:(r + 1) * _RB, :]
    q = jax.lax.dot_general(xr, wq_ref[...],
                            (((1,), (1,)), ((), ())),
                            preferred_element_type=jnp.float32)
    logits = jax.lax.dot_general(q, kt,
                                 (((1,), (0,)), ((), ())),
                                 preferred_element_type=jnp.float32) * scale
    m = jnp.max(logits, axis=-1, keepdims=True)
    e = jnp.exp(logits - m)
    return e / jnp.sum(e, axis=-1, keepdims=True)


def _make_kt(x_ref, wk_ref):
    return jax.lax.dot_general(wk_ref[...], x_ref[0],
                               (((1,), (1,)), ((), ())),
                               preferred_element_type=jnp.float32)


def _colmask_kernel(x_ref, wq_ref, wk_ref, cm_ref, se_ref):
    h = pl.program_id(1)
    N = se_ref.shape[-1]
    C = x_ref.shape[-1]
    scale = C ** -0.5
    kt = _make_kt(x_ref, wk_ref)
    for r in range(N // _RB):
        attn = _attn_rows(x_ref, wq_ref, wk_ref, kt, r, scale)
        sl = pl.ds(r * _RB, _RB)

        @pl.when(h == 0)
        def _():
            se_ref[sl, :] = attn

        @pl.when(h > 0)
        def _():
            se_ref[sl, :] = se_ref[sl, :] + attn

    @pl.when(h == _H - 1)
    def _():
        cm = jnp.zeros((1, N), jnp.float32)
        for r in range(N // _RB):
            se = se_ref[r * _RB:(r + 1) * _RB, :]
            col = jax.lax.broadcasted_iota(jnp.int32, (_RB, N), 1)
            for _ in range(_NB):
                v = jnp.max(se, axis=-1, keepdims=True)
                first = jnp.min(jnp.where(se == v, col, N),
                                axis=-1, keepdims=True)
                onehot = col == first
                hit = jnp.max(jnp.where(onehot, 1.0, 0.0),
                              axis=0, keepdims=True)
                cm = jnp.maximum(cm, hit)
                se = jnp.where(onehot, -1.0, se)
        cm_ref[...] = cm.reshape(1, 1, N)


def _edge_kernel(x_ref, wq_ref, wk_ref, cm_ref, out_ref, nr_ref):
    N = nr_ref.shape[-1]
    C = x_ref.shape[-1]
    scale = C ** -0.5
    cm = cm_ref[0]
    kt = _make_kt(x_ref, wk_ref)
    s = jnp.zeros((1, N), jnp.float32)
    for r in range(N // _RB):
        attn = _attn_rows(x_ref, wq_ref, wk_ref, kt, r, scale)
        col = jax.lax.broadcasted_iota(jnp.int32, (_RB, N), 1)
        row = jax.lax.broadcasted_iota(jnp.int32, (_RB, N), 0) + r * _RB
        ne = jnp.where(col == row, attn, attn * cm)
        rs = jnp.sum(ne, axis=-1, keepdims=True) + 1e-16
        nr = ne / rs
        nr_ref[r * _RB:(r + 1) * _RB, :] = nr
        s = s + jnp.sum(nr, axis=0, keepdims=True)
    sinv = 1.0 / (s + 1e-16)
    for r in range(N // _RB):
        a = nr_ref[r * _RB:(r + 1) * _RB, :] * sinv
        out_ref[0, 0, r * _RB:(r + 1) * _RB, :] = jax.lax.dot_general(
            a, nr_ref[...], (((1,), (1,)), ((), ())),
            preferred_element_type=jnp.float32)


@jax.jit
def kernel(x, W_qk):
    B, N, C = x.shape
    f32 = jnp.float32

    colmask = pl.pallas_call(
        _colmask_kernel,
        grid=(B, _H),
        in_specs=[
            pl.BlockSpec((1, N, C), lambda b, h: (b, 0, 0)),
            pl.BlockSpec((C, C), lambda b, h: (h, 0)),
            pl.BlockSpec((C, C), lambda b, h: (_H + h, 0)),
        ],
        out_specs=pl.BlockSpec((1, 1, N), lambda b, h: (b, 0, 0)),
        out_shape=jax.ShapeDtypeStruct((B, 1, N), f32),
        scratch_shapes=[pltpu.VMEM((N, N), f32)],
        compiler_params=pltpu.CompilerParams(
            dimension_semantics=("parallel", "arbitrary")),
    )(x, W_qk, W_qk)

    out = pl.pallas_call(
        _edge_kernel,
        grid=(B, _H),
        in_specs=[
            pl.BlockSpec((1, N, C), lambda b, h: (b, 0, 0)),
            pl.BlockSpec((C, C), lambda b, h: (h, 0)),
            pl.BlockSpec((C, C), lambda b, h: (_H + h, 0)),
            pl.BlockSpec((1, 1, N), lambda b, h: (b, 0, 0)),
        ],
        out_specs=pl.BlockSpec((1, 1, N, N), lambda b, h: (b, h, 0, 0)),
        out_shape=jax.ShapeDtypeStruct((B, _H, N, N), f32),
        scratch_shapes=[pltpu.VMEM((N, N), f32)],
        compiler_params=pltpu.CompilerParams(
            dimension_semantics=("parallel", "arbitrary")),
    )(x, W_qk, W_qk, colmask)
    return out


# speedup vs baseline: 4.0341x; 1.0248x over previous
import functools

import jax
import jax.numpy as jnp
from jax.experimental import pallas as pl
from jax.experimental.pallas import tpu as pltpu

_H = 8
_NB = 5
_RB = 256


def _attn_rows(x_ref, wq_ref, wk_ref, kt, r, scale):
    xr = x_ref[0, r * _RB:(r + 1) * _RB, :]
    q = jax.lax.dot_general(xr, wq_ref[...],
                            (((1,), (1,)), ((), ())),
                            preferred_element_type=jnp.float32)
    logits = jax.lax.dot_general(q, kt,
                                 (((1,), (0,)), ((), ())),
                                 preferred_element_type=jnp.float32) * scale
    m = jnp.max(logits, axis=-1, keepdims=True)
    e = jnp.exp(logits - m)
    return e / jnp.sum(e, axis=-1, keepdims=True)


def _make_kt(x_ref, wk_ref):
    return jax.lax.dot_general(wk_ref[...], x_ref[0],
                               (((1,), (1,)), ((), ())),
                               preferred_element_type=jnp.float32)


def _colmask_kernel(x_ref, wq_ref, wk_ref, cm_ref, se_ref):
    h = pl.program_id(1)
    N = se_ref.shape[-1]
    C = x_ref.shape[-1]
    scale = C ** -0.5
    kt = _make_kt(x_ref, wk_ref)
    for r in range(N // _RB):
        attn = _attn_rows(x_ref, wq_ref, wk_ref, kt, r, scale)
        sl = pl.ds(r * _RB, _RB)

        @pl.when(h == 0)
        def _():
            se_ref[sl, :] = attn

        @pl.when(h > 0)
        def _():
            se_ref[sl, :] = se_ref[sl, :] + attn

    @pl.when(h == _H - 1)
    def _():
        cm = jnp.zeros((1, N), jnp.float32)
        for r in range(N // _RB):
            se = se_ref[r * _RB:(r + 1) * _RB, :]
            col = jax.lax.broadcasted_iota(jnp.int32, (_RB, N), 1)
            for _ in range(_NB):
                idx = jnp.argmax(se, axis=-1, keepdims=True)
                se = jnp.where(col == idx, -1.0, se)
            hit = jnp.max(jnp.where(se == -1.0, 1.0, 0.0),
                          axis=0, keepdims=True)
            cm = jnp.maximum(cm, hit)
        cm_ref[...] = cm.reshape(1, 1, N)


def _edge_kernel(x_ref, wq_ref, wk_ref, cm_ref, out_ref, nr_ref):
    N = nr_ref.shape[-1]
    C = x_ref.shape[-1]
    scale = C ** -0.5
    cm = cm_ref[0]
    kt = _make_kt(x_ref, wk_ref)
    s = jnp.zeros((1, N), jnp.float32)
    for r in range(N // _RB):
        attn = _attn_rows(x_ref, wq_ref, wk_ref, kt, r, scale)
        col = jax.lax.broadcasted_iota(jnp.int32, (_RB, N), 1)
        row = jax.lax.broadcasted_iota(jnp.int32, (_RB, N), 0) + r * _RB
        ne = jnp.where(col == row, attn, attn * cm)
        rs = jnp.sum(ne, axis=-1, keepdims=True) + 1e-16
        nr = ne / rs
        nr_ref[r * _RB:(r + 1) * _RB, :] = nr.astype(jnp.bfloat16)
        s = s + jnp.sum(nr, axis=0, keepdims=True)
    sinv = (1.0 / (s + 1e-16)).astype(jnp.bfloat16)
    for r in range(N // _RB):
        a = nr_ref[r * _RB:(r + 1) * _RB, :] * sinv
        out_ref[0, 0, r * _RB:(r + 1) * _RB, :] = jax.lax.dot_general(
            a, nr_ref[...], (((1,), (1,)), ((), ())),
            preferred_element_type=jnp.float32)


@jax.jit
def kernel(x, W_qk):
    B, N, C = x.shape
    f32 = jnp.float32

    colmask = pl.pallas_call(
        _colmask_kernel,
        grid=(B, _H),
        in_specs=[
            pl.BlockSpec((1, N, C), lambda b, h: (b, 0, 0)),
            pl.BlockSpec((C, C), lambda b, h: (h, 0)),
            pl.BlockSpec((C, C), lambda b, h: (_H + h, 0)),
        ],
        out_specs=pl.BlockSpec((1, 1, N), lambda b, h: (b, 0, 0)),
        out_shape=jax.ShapeDtypeStruct((B, 1, N), f32),
        scratch_shapes=[pltpu.VMEM((N, N), f32)],
        compiler_params=pltpu.CompilerParams(
            dimension_semantics=("parallel", "arbitrary")),
    )(x, W_qk, W_qk)

    out = pl.pallas_call(
        _edge_kernel,
        grid=(B, _H),
        in_specs=[
            pl.BlockSpec((1, N, C), lambda b, h: (b, 0, 0)),
            pl.BlockSpec((C, C), lambda b, h: (h, 0)),
            pl.BlockSpec((C, C), lambda b, h: (_H + h, 0)),
            pl.BlockSpec((1, 1, N), lambda b, h: (b, 0, 0)),
        ],
        out_specs=pl.BlockSpec((1, 1, N, N), lambda b, h: (b, h, 0, 0)),
        out_shape=jax.ShapeDtypeStruct((B, _H, N, N), f32),
        scratch_shapes=[pltpu.VMEM((N, N), jnp.bfloat16)],
        compiler_params=pltpu.CompilerParams(
            dimension_semantics=("parallel", "arbitrary")),
    )(x, W_qk, W_qk, colmask)
    return out
